# Initial kernel scaffold; baseline (speedup 1.0000x reference)
import jax
import jax.numpy as jnp
from jax.experimental import pallas as pl
from jax.experimental.pallas import tpu as pltpu
from jax.experimental.pallas import tpu_sc as plsc

_DIM = 512
_WINDOW = 64


def kernel(x, weight):
    batch, seq = x.shape
    n = batch * seq
    idx = x.reshape(1, n)
    mesh = plsc.VectorSubcoreMesh(core_axis_name="core", subcore_axis_name="subcore")

    @pl.kernel(
        out_type=jax.ShapeDtypeStruct((n, _DIM), weight.dtype),
        mesh=mesh,
    )
    def gather_kernel(w_hbm, i_hbm, o_hbm):
        def body(i_vmem, o_vmem):
            pltpu.sync_copy(w_hbm.at[i_vmem.at[0]], o_vmem)

        pltpu.emit_pipeline(
            body,
            grid=(n // _WINDOW,),
            in_specs=[pl.BlockSpec((1, _WINDOW), index_map=lambda i: (0, i))],
            out_specs=[pl.BlockSpec((_WINDOW, _DIM), index_map=lambda i: (i, 0))],
            core_axis_name=("core", "subcore"),
            dimension_semantics=(pltpu.PARALLEL,),
        )(i_hbm, o_hbm)

    out = gather_kernel(weight, idx)
    return out.reshape(batch, seq, _DIM)


# speedup vs baseline: 1.6862x; 1.6862x over previous
import jax
import jax.numpy as jnp
from jax.experimental import pallas as pl
from jax.experimental.pallas import tpu as pltpu
from jax.experimental.pallas import tpu_sc as plsc

_SPLIT = 2
_WINDOW = 128


def kernel(x, weight):
    batch, seq = x.shape
    vocab, dim = weight.shape
    sub = dim // _SPLIT
    n = batch * seq * _SPLIT

    w2 = weight.reshape(vocab * _SPLIT, sub)
    idx = (x.reshape(-1, 1) * _SPLIT
           + jnp.arange(_SPLIT, dtype=x.dtype)).reshape(n // _WINDOW, _WINDOW)

    mesh = plsc.VectorSubcoreMesh(core_axis_name="core", subcore_axis_name="subcore")

    @pl.kernel(
        out_type=jax.ShapeDtypeStruct((n, sub), weight.dtype),
        mesh=mesh,
    )
    def gather_kernel(w_hbm, i_hbm, o_hbm):
        def body(i_vmem, o_vmem):
            pltpu.sync_copy(w_hbm.at[i_vmem.at[0]], o_vmem)

        pltpu.emit_pipeline(
            body,
            grid=(n // _WINDOW,),
            in_specs=[pl.BlockSpec((1, _WINDOW), index_map=lambda i: (i, 0))],
            out_specs=[pl.BlockSpec((_WINDOW, sub), index_map=lambda i: (i, 0))],
            core_axis_name=("core", "subcore"),
            dimension_semantics=(pltpu.PARALLEL,),
        )(i_hbm, o_hbm)

    out = gather_kernel(w2, idx)
    return out.reshape(batch, seq, dim)


# speedup vs baseline: 2.0072x; 1.1904x over previous
import jax
import jax.numpy as jnp
from jax.experimental import pallas as pl
from jax.experimental.pallas import tpu as pltpu
from jax.experimental.pallas import tpu_sc as plsc

_NC = 2
_NS = 16
_NW = _NC * _NS
_CHUNK = 32
_NBUF = 4
_BB = 64


def _sc_gather(x_3d, weight):
    _, chunks, _ = x_3d.shape
    n = x_3d.size
    _, dim = weight.shape
    toks_w = n // _NW

    mesh = plsc.VectorSubcoreMesh(core_axis_name="c", subcore_axis_name="s")

    @pl.kernel(
        out_type=jax.ShapeDtypeStruct((n, dim), weight.dtype),
        mesh=mesh,
        scratch_types=(
            [pltpu.VMEM((chunks, _CHUNK), jnp.int32)]
            + [pltpu.VMEM((_CHUNK, dim), jnp.float32) for _ in range(_NBUF)]
            + [pltpu.SemaphoreType.DMA for _ in range(2 * _NBUF)]
        ),
    )
    def gather_kernel(w_hbm, i_hbm, o_hbm, idx_v, *bufs_and_sems):
        bufs = bufs_and_sems[:_NBUF]
        gsem = bufs_and_sems[_NBUF:2 * _NBUF]
        wsem = bufs_and_sems[2 * _NBUF:]

        wid = jax.lax.axis_index("s") * _NC + jax.lax.axis_index("c")
        tok0 = wid * toks_w

        pltpu.sync_copy(i_hbm.at[wid], idx_v)

        def start_gather(c, b):
            pltpu.async_copy(w_hbm.at[idx_v.at[c]], bufs[b], gsem[b])

        def wait_gather(b):
            pltpu.make_async_copy(
                w_hbm.at[idx_v.at[0]], bufs[b], gsem[b]).wait()

        def start_write(c, b):
            pltpu.async_copy(
                bufs[b], o_hbm.at[pl.ds(tok0 + c * _CHUNK, _CHUNK)], wsem[b])

        def wait_write(b):
            pltpu.make_async_copy(
                bufs[b], o_hbm.at[pl.ds(tok0, _CHUNK)], wsem[b]).wait()

        for b in range(_NBUF):
            start_gather(b, b)

        @pl.loop(0, chunks, step=_NBUF)
        def _(c0):
            for b in range(_NBUF):
                c = c0 + b
                wait_gather(b)
                start_write(c, b)
                nxt = c + _NBUF

                @pl.when(nxt < chunks)
                def _():
                    wait_write(b)
                    start_gather(nxt, b)

        for b in range(_NBUF):
            wait_write(b)

    return gather_kernel(weight, x_3d)


def _tc_relayout(dense, batch, seq, dim):
    def body(i_ref, o_ref):
        o_ref[...] = i_ref[...].reshape(_BB, seq, dim)

    return pl.pallas_call(
        body,
        grid=(batch // _BB,),
        in_specs=[pl.BlockSpec((_BB * seq, dim), lambda i: (i, 0))],
        out_specs=pl.BlockSpec((_BB, seq, dim), lambda i: (i, 0, 0)),
        out_shape=jax.ShapeDtypeStruct((batch, seq, dim), dense.dtype),
        compiler_params=pltpu.CompilerParams(
            dimension_semantics=("parallel",)),
    )(dense)


def kernel(x, weight):
    batch, seq = x.shape
    _, dim = weight.shape
    n = batch * seq
    toks_w = n // _NW
    dense = _sc_gather(x.reshape(_NW, toks_w // _CHUNK, _CHUNK), weight)
    return _tc_relayout(dense, batch, seq, dim)
